# baseline (speedup 1.0000x reference)
import jax
import jax.numpy as jnp
from jax import lax
from jax.experimental import pallas as pl
from jax.experimental.pallas import tpu as pltpu
from jax.experimental.pallas import tpu_sc as plsc

N_USERS = 50000
LATDIM = 64
DFUSED = 128
E_TOTAL = 1600000

NC, NS, LANES = 2, 16, 16
NW = NC * NS
CHUNK = 128
N_CHUNKS = E_TOTAL // CHUNK
BASE_CHUNKS = N_CHUNKS // NW
EXTRA_CHUNKS = N_CHUNKS % NW
ROW_BLK = 2000
RC_BLK = 500
NSLOT = 3


def _pack_body(rows_ref, cols_ref, rc_ref):
    rc_ref[...] = jnp.bitwise_or(rows_ref[...],
                                 lax.shift_left(cols_ref[...], 16))


def _pack_ids(rows2d, cols2d):
    return pl.pallas_call(
        _pack_body,
        out_shape=jax.ShapeDtypeStruct((N_CHUNKS, 128), jnp.int32),
    )(rows2d, cols2d)


def _prep_body(ukey_ref, uemb_ref, ikey_ref, iemb_ref, uh_ref, ih_ref,
               a_ref, b_ref):
    m = lax.dot_general(uh_ref[...], ih_ref[...], (((1,), (1,)), ((), ())),
                        preferred_element_type=jnp.float32,
                        precision=lax.Precision.HIGHEST)
    a_ref[:, 0:LATDIM] = lax.dot_general(
        ukey_ref[...], m, (((1,), (0,)), ((), ())),
        preferred_element_type=jnp.float32, precision=lax.Precision.HIGHEST)
    a_ref[:, LATDIM:DFUSED] = uemb_ref[...]
    b_ref[:, 0:LATDIM] = ikey_ref[...]
    b_ref[:, LATDIM:DFUSED] = iemb_ref[...]


def _prep_tables(ukey, uemb, ikey, iemb, uh, ih):
    grid = N_USERS // ROW_BLK
    row_spec = pl.BlockSpec((ROW_BLK, LATDIM), lambda i: (i, 0))
    hyper_spec = pl.BlockSpec((LATDIM, 128), lambda i: (0, 0))
    out_spec = pl.BlockSpec((ROW_BLK, DFUSED), lambda i: (i, 0))
    return pl.pallas_call(
        _prep_body,
        grid=(grid,),
        in_specs=[row_spec, row_spec, row_spec, row_spec,
                  hyper_spec, hyper_spec],
        out_specs=[out_spec, out_spec],
        out_shape=[
            jax.ShapeDtypeStruct((N_USERS, DFUSED), jnp.float32),
            jax.ShapeDtypeStruct((N_USERS, DFUSED), jnp.float32),
        ],
    )(ukey, uemb, ikey, iemb, uh, ih)


def _sc_body(a_hbm, b_hbm, rc_hbm, eids_hbm, out_hbm,
             eids_v, rc_v, u_v, i_v, a_v, b_v, ta_v, tb_v, out_v,
             sem_e0, sem_e1, sem_e2, sem_rc0, sem_rc1, sem_rc2,
             sem_ab0, sem_ab1, sem_ab2, sem_o0, sem_o1, sem_o2):
    cid = lax.axis_index("c")
    sid = lax.axis_index("s")
    wid = sid * NC + cid
    my_n = BASE_CHUNKS + jnp.where(wid < EXTRA_CHUNKS, 1, 0)
    sem_e = (sem_e0, sem_e1, sem_e2)
    sem_rc = (sem_rc0, sem_rc1, sem_rc2)
    sem_ab = (sem_ab0, sem_ab1, sem_ab2)
    sem_o = (sem_o0, sem_o1, sem_o2)

    def goff(j):
        return (wid + j * NW) * CHUNK

    def issue_eids(j, s):
        return pltpu.async_copy(eids_hbm.at[pl.ds(goff(j), CHUNK)],
                                eids_v.at[s], sem_e[s])

    def issue_rc(s):
        return pltpu.async_copy(rc_hbm.at[eids_v.at[s]], rc_v.at[s],
                                sem_rc[s])

    def issue_ab(s):
        pltpu.async_copy(a_hbm.at[u_v.at[s]], a_v.at[s], sem_ab[s])
        pltpu.async_copy(b_hbm.at[i_v.at[s]], b_v.at[s], sem_ab[s])

    def wait_eids(s):
        pltpu.make_async_copy(eids_hbm.at[pl.ds(0, CHUNK)], eids_v.at[s],
                              sem_e[s]).wait()

    def wait_rc(s):
        pltpu.make_async_copy(eids_hbm.at[pl.ds(0, CHUNK)], rc_v.at[s],
                              sem_rc[s]).wait()

    def wait_ab(s):
        pltpu.make_async_copy(a_hbm.at[pl.ds(0, CHUNK)], a_v.at[s],
                              sem_ab[s]).wait()
        pltpu.make_async_copy(b_hbm.at[pl.ds(0, CHUNK)], b_v.at[s],
                              sem_ab[s]).wait()

    def wait_out(s):
        pltpu.make_async_copy(eids_hbm.at[pl.ds(0, CHUNK)], out_v.at[s],
                              sem_o[s]).wait()

    def unpack_ids(s):
        for g in range(CHUNK // LANES):
            sl = pl.ds(g * LANES, LANES)
            rc = rc_v[s, sl]
            u_v[s, sl] = jnp.bitwise_and(rc, 0xFFFF)
            i_v[s, sl] = lax.shift_right_logical(rc, 16)

    def compute_chunk(k, s):
        col17 = lax.iota(jnp.int32, LANES) * 17

        @pl.when(k >= NSLOT)
        def _drain_prev_out():
            wait_out(s)

        TILE = LANES * 17

        def scatter_group(g, off):
            for j in range(LANES):
                e = g * LANES + j

                def prod(c):
                    return (a_v[s, e, pl.ds(c * LANES, LANES)]
                            * b_v[s, e, pl.ds(c * LANES, LANES)])

                pa = (prod(0) + prod(1)) + (prod(2) + prod(3))
                pb = (prod(4) + prod(5)) + (prod(6) + prod(7))
                idx = col17 + (off + j)
                plsc.store_scatter(ta_v, [idx], pa)
                plsc.store_scatter(tb_v, [idx], pb)

        def flush_group(g, off):
            ra = [ta_v[pl.ds(off + l * 17, LANES)] for l in range(LANES)]
            rb = [tb_v[pl.ds(off + l * 17, LANES)] for l in range(LANES)]
            while len(ra) > 1:
                ra = [x + y for x, y in zip(ra[::2], ra[1::2])]
                rb = [x + y for x, y in zip(rb[::2], rb[1::2])]
            score = 1.0 / (1.0 + jnp.exp(-ra[0]))
            out_v[s, pl.ds(g * LANES, LANES)] = jnp.abs(score - rb[0])

        def pair_grp_body(q, _):
            g0 = 2 * q
            g1 = g0 + 1
            scatter_group(g0, 0)
            scatter_group(g1, TILE)
            flush_group(g0, 0)
            flush_group(g1, TILE)
            return _

        lax.fori_loop(0, CHUNK // LANES // 2, pair_grp_body, None)
        pltpu.async_copy(out_v.at[s], out_hbm.at[pl.ds(goff(k), CHUNK)],
                         sem_o[s])

    def step(k, s):
        s1, s2 = (s + 1) % NSLOT, (s + 2) % NSLOT
        wait_ab(s)

        @pl.when(k + 2 < my_n)
        def _start_rc():
            wait_eids(s2)
            issue_rc(s2)

        @pl.when(k + 1 < my_n)
        def _start_ab():
            wait_rc(s1)
            unpack_ids(s1)
            issue_ab(s1)

        @pl.when(k + 4 < my_n)
        def _prefetch_eids():
            issue_eids(k + 4, s1)

        compute_chunk(k, s)

    issue_eids(0, 0).wait()
    issue_rc(0)
    issue_eids(1, 1).wait()
    wait_rc(0)
    unpack_ids(0)
    issue_ab(0)
    issue_rc(1)
    issue_eids(2, 2)
    issue_eids(3, 0)

    def trip_body(t, _):
        k0 = 3 * t
        step(k0, 0)

        @pl.when(k0 + 1 < my_n)
        def _s1():
            step(k0 + 1, 1)

        @pl.when(k0 + 2 < my_n)
        def _s2():
            step(k0 + 2, 2)

        return _

    lax.fori_loop(0, (my_n + 2) // NSLOT, trip_body, None)
    for s in range(NSLOT):
        wait_out(s)


def _edge_scores(a_tab, b_tab, rc_tab, edgeids):
    mesh = plsc.VectorSubcoreMesh(core_axis_name="c", subcore_axis_name="s",
                                  num_cores=NC, num_subcores=NS)
    f = pl.kernel(
        _sc_body,
        out_type=jax.ShapeDtypeStruct((E_TOTAL,), jnp.float32),
        mesh=mesh,
        compiler_params=pltpu.CompilerParams(needs_layout_passes=False),
        scratch_types=[
            pltpu.VMEM((NSLOT, CHUNK), jnp.int32),
            pltpu.VMEM((NSLOT, CHUNK), jnp.int32),
            pltpu.VMEM((NSLOT, CHUNK), jnp.int32),
            pltpu.VMEM((NSLOT, CHUNK), jnp.int32),
            pltpu.VMEM((NSLOT, CHUNK, DFUSED), jnp.float32),
            pltpu.VMEM((NSLOT, CHUNK, DFUSED), jnp.float32),
            pltpu.VMEM((2 * LANES * 17,), jnp.float32),
            pltpu.VMEM((2 * LANES * 17,), jnp.float32),
            pltpu.VMEM((NSLOT, CHUNK), jnp.float32),
        ] + [pltpu.SemaphoreType.DMA] * 12,
    )
    return f(a_tab, b_tab, rc_tab, edgeids)


def kernel(ui_uKey, ui_iKey, uEmbeds, iEmbeds, ui_uHyper, ui_iHyper,
           rows, cols, edgeids):
    ukey = jnp.transpose(ui_uKey, (1, 0, 2)).reshape(-1, LATDIM)
    ikey = jnp.transpose(ui_iKey, (1, 0, 2)).reshape(-1, LATDIM)
    rows2d = rows.reshape(N_CHUNKS, 128)
    cols2d = cols.reshape(N_CHUNKS, 128)
    a_tab, b_tab = _prep_tables(ukey, uEmbeds, ikey, iEmbeds,
                                ui_uHyper, ui_iHyper)
    rc2d = _pack_ids(rows2d, cols2d)
    return _edge_scores(a_tab, b_tab, rc2d.reshape(E_TOTAL), edgeids)


# speedup vs baseline: 1.3154x; 1.3154x over previous
import jax
import jax.numpy as jnp
from jax import lax
from jax.experimental import pallas as pl
from jax.experimental.pallas import tpu as pltpu
from jax.experimental.pallas import tpu_sc as plsc

N_USERS = 50000
LATDIM = 64
DFUSED = 128
E_TOTAL = 1600000

NC, NS, LANES = 2, 16, 16
NW = NC * NS
CHUNK = 128
N_CHUNKS = E_TOTAL // CHUNK
BASE_CHUNKS = N_CHUNKS // NW
EXTRA_CHUNKS = N_CHUNKS % NW
ROW_BLK = 2000
RC_BLK = 500
NSLOT = 3


def _pack_body(rows_ref, cols_ref, rc_ref):
    rc_ref[...] = jnp.bitwise_or(rows_ref[...],
                                 lax.shift_left(cols_ref[...], 16))


def _pack_ids(rows2d, cols2d):
    return pl.pallas_call(
        _pack_body,
        out_shape=jax.ShapeDtypeStruct((N_CHUNKS, 128), jnp.int32),
    )(rows2d, cols2d)


def _prep_body(ukey_ref, uemb_ref, ikey_ref, iemb_ref, uh_ref, ih_ref,
               a_ref, b_ref):
    m = lax.dot_general(uh_ref[...], ih_ref[...], (((1,), (1,)), ((), ())),
                        preferred_element_type=jnp.float32,
                        precision=lax.Precision.HIGHEST)
    a_ref[:, 0:LATDIM] = lax.dot_general(
        ukey_ref[...], m, (((1,), (0,)), ((), ())),
        preferred_element_type=jnp.float32, precision=lax.Precision.HIGHEST)
    a_ref[:, LATDIM:DFUSED] = uemb_ref[...]
    b_ref[:, 0:LATDIM] = ikey_ref[...]
    b_ref[:, LATDIM:DFUSED] = iemb_ref[...]


def _prep_tables(ukey, uemb, ikey, iemb, uh, ih):
    grid = N_USERS // ROW_BLK
    row_spec = pl.BlockSpec((ROW_BLK, LATDIM), lambda i: (i, 0))
    hyper_spec = pl.BlockSpec((LATDIM, 128), lambda i: (0, 0))
    out_spec = pl.BlockSpec((ROW_BLK, DFUSED), lambda i: (i, 0))
    return pl.pallas_call(
        _prep_body,
        grid=(grid,),
        in_specs=[row_spec, row_spec, row_spec, row_spec,
                  hyper_spec, hyper_spec],
        out_specs=[out_spec, out_spec],
        out_shape=[
            jax.ShapeDtypeStruct((N_USERS, DFUSED), jnp.float32),
            jax.ShapeDtypeStruct((N_USERS, DFUSED), jnp.float32),
        ],
    )(ukey, uemb, ikey, iemb, uh, ih)


def _sc_body(a_hbm, b_hbm, rc_hbm, eids_hbm, out_hbm,
             eids_v, rc_v, u_v, i_v, a_v, b_v, ta_v, tb_v, out_v,
             sem_e0, sem_e1, sem_e2, sem_rc0, sem_rc1, sem_rc2,
             sem_ab0, sem_ab1, sem_ab2, sem_o0, sem_o1, sem_o2):
    cid = lax.axis_index("c")
    sid = lax.axis_index("s")
    wid = sid * NC + cid
    my_n = BASE_CHUNKS + jnp.where(wid < EXTRA_CHUNKS, 1, 0)
    sem_e = (sem_e0, sem_e1, sem_e2)
    sem_rc = (sem_rc0, sem_rc1, sem_rc2)
    sem_ab = (sem_ab0, sem_ab1, sem_ab2)
    sem_o = (sem_o0, sem_o1, sem_o2)

    def goff(j):
        return (wid + j * NW) * CHUNK

    def issue_eids(j, s):
        return pltpu.async_copy(eids_hbm.at[pl.ds(goff(j), CHUNK)],
                                eids_v.at[s], sem_e[s])

    def issue_rc(s):
        return pltpu.async_copy(rc_hbm.at[eids_v.at[s]], rc_v.at[s],
                                sem_rc[s])

    def issue_ab(s):
        pltpu.async_copy(a_hbm.at[u_v.at[s]], a_v.at[s], sem_ab[s])
        pltpu.async_copy(b_hbm.at[i_v.at[s]], b_v.at[s], sem_ab[s])

    def wait_eids(s):
        pltpu.make_async_copy(eids_hbm.at[pl.ds(0, CHUNK)], eids_v.at[s],
                              sem_e[s]).wait()

    def wait_rc(s):
        pltpu.make_async_copy(eids_hbm.at[pl.ds(0, CHUNK)], rc_v.at[s],
                              sem_rc[s]).wait()

    def wait_ab(s):
        pltpu.make_async_copy(a_hbm.at[pl.ds(0, CHUNK)], a_v.at[s],
                              sem_ab[s]).wait()
        pltpu.make_async_copy(b_hbm.at[pl.ds(0, CHUNK)], b_v.at[s],
                              sem_ab[s]).wait()

    def wait_out(s):
        pltpu.make_async_copy(eids_hbm.at[pl.ds(0, CHUNK)], out_v.at[s],
                              sem_o[s]).wait()

    def unpack_ids(s):
        for g in range(CHUNK // LANES):
            sl = pl.ds(g * LANES, LANES)
            rc = rc_v[s, sl]
            u_v[s, sl] = jnp.bitwise_and(rc, 0xFFFF)
            i_v[s, sl] = lax.shift_right_logical(rc, 16)

    def compute_chunk(k, s):
        col17 = lax.iota(jnp.int32, LANES) * 17

        @pl.when(k >= NSLOT)
        def _drain_prev_out():
            wait_out(s)

        def group_body(g, _):
            def edge_pp(j):
                e = g * LANES + j

                def prod(c):
                    return (a_v[s, e, pl.ds(c * LANES, LANES)]
                            * b_v[s, e, pl.ds(c * LANES, LANES)])

                pa = ((prod(0) + prod(1)) + prod(2)) + prod(3)
                pb = ((prod(4) + prod(5)) + prod(6)) + prod(7)
                idx = col17 + j
                plsc.store_scatter(ta_v, [idx], pa)
                plsc.store_scatter(tb_v, [idx], pb)

            for j in range(LANES):
                edge_pp(j)
            ra = [ta_v[pl.ds(l * 17, LANES)] for l in range(LANES)]
            rb = [tb_v[pl.ds(l * 17, LANES)] for l in range(LANES)]
            while len(ra) > 1:
                ra = [x + y for x, y in zip(ra[::2], ra[1::2])]
                rb = [x + y for x, y in zip(rb[::2], rb[1::2])]
            suma, sumb = ra[0], rb[0]
            score = 1.0 / (1.0 + jnp.exp(-suma))
            gsl = pl.ds(g * LANES, LANES)
            out_v[s, gsl] = jnp.abs(score - sumb)
            return _

        lax.fori_loop(0, CHUNK // LANES, group_body, None)
        pltpu.async_copy(out_v.at[s], out_hbm.at[pl.ds(goff(k), CHUNK)],
                         sem_o[s])

    def step(k, s):
        s1, s2 = (s + 1) % NSLOT, (s + 2) % NSLOT
        wait_ab(s)

        @pl.when(k + 2 < my_n)
        def _start_rc():
            wait_eids(s2)
            issue_rc(s2)

        @pl.when(k + 1 < my_n)
        def _start_ab():
            wait_rc(s1)
            unpack_ids(s1)
            issue_ab(s1)

        @pl.when(k + 4 < my_n)
        def _prefetch_eids():
            issue_eids(k + 4, s1)

        compute_chunk(k, s)

    issue_eids(0, 0).wait()
    issue_rc(0)
    issue_eids(1, 1).wait()
    wait_rc(0)
    unpack_ids(0)
    issue_ab(0)
    issue_rc(1)
    issue_eids(2, 2)
    issue_eids(3, 0)

    def trip_body(t, _):
        k0 = 3 * t
        step(k0, 0)

        @pl.when(k0 + 1 < my_n)
        def _s1():
            step(k0 + 1, 1)

        @pl.when(k0 + 2 < my_n)
        def _s2():
            step(k0 + 2, 2)

        return _

    lax.fori_loop(0, (my_n + 2) // NSLOT, trip_body, None)
    for s in range(NSLOT):
        wait_out(s)


def _edge_scores(a_tab, b_tab, rc_tab, edgeids):
    mesh = plsc.VectorSubcoreMesh(core_axis_name="c", subcore_axis_name="s",
                                  num_cores=NC, num_subcores=NS)
    f = pl.kernel(
        _sc_body,
        out_type=jax.ShapeDtypeStruct((E_TOTAL,), jnp.float32),
        mesh=mesh,
        compiler_params=pltpu.CompilerParams(needs_layout_passes=False),
        scratch_types=[
            pltpu.VMEM((NSLOT, CHUNK), jnp.int32),
            pltpu.VMEM((NSLOT, CHUNK), jnp.int32),
            pltpu.VMEM((NSLOT, CHUNK), jnp.int32),
            pltpu.VMEM((NSLOT, CHUNK), jnp.int32),
            pltpu.VMEM((NSLOT, CHUNK, DFUSED), jnp.float32),
            pltpu.VMEM((NSLOT, CHUNK, DFUSED), jnp.float32),
            pltpu.VMEM((LANES * 17,), jnp.float32),
            pltpu.VMEM((LANES * 17,), jnp.float32),
            pltpu.VMEM((NSLOT, CHUNK), jnp.float32),
        ] + [pltpu.SemaphoreType.DMA] * 12,
    )
    return f(a_tab, b_tab, rc_tab, edgeids)


def kernel(ui_uKey, ui_iKey, uEmbeds, iEmbeds, ui_uHyper, ui_iHyper,
           rows, cols, edgeids):
    ukey = jnp.transpose(ui_uKey, (1, 0, 2)).reshape(-1, LATDIM)
    ikey = jnp.transpose(ui_iKey, (1, 0, 2)).reshape(-1, LATDIM)
    rows2d = rows.reshape(N_CHUNKS, 128)
    cols2d = cols.reshape(N_CHUNKS, 128)
    a_tab, b_tab = _prep_tables(ukey, uEmbeds, ikey, iEmbeds,
                                ui_uHyper, ui_iHyper)
    rc2d = _pack_ids(rows2d, cols2d)
    return _edge_scores(a_tab, b_tab, rc2d.reshape(E_TOTAL), edgeids)
